# baseline (speedup 1.0000x reference)
import jax
import jax.numpy as jnp
from jax.experimental import pallas as pl
from jax.experimental.pallas import tpu as pltpu

_CHUNKS = [256, 256, 512] + [1024] * 15
_OFFS = [sum(_CHUNKS[:j]) for j in range(len(_CHUNKS))]
_NSLOT = 3
_MAXC = 1024


def _gate_body(x_hbm, w_ref, o_hbm, xbuf, obuf, isems, osems):
    n = len(_CHUNKS)

    def in_copy(j):
        s = j % _NSLOT
        return pltpu.make_async_copy(
            x_hbm.at[pl.ds(_OFFS[j], _CHUNKS[j]), :],
            xbuf.at[s, pl.ds(0, _CHUNKS[j]), :],
            isems.at[s],
        )

    def out_copy(j):
        s = j % _NSLOT
        return pltpu.make_async_copy(
            obuf.at[s, :, pl.ds(0, _CHUNKS[j])],
            o_hbm.at[:, pl.ds(_OFFS[j], _CHUNKS[j])],
            osems.at[s],
        )

    for j in range(min(_NSLOT, n)):
        in_copy(j).start()

    for j in range(n):
        s = j % _NSLOT
        in_copy(j).wait()
        if j >= _NSLOT:
            out_copy(j - _NSLOT).wait()
        obuf[s, :, pl.ds(0, _CHUNKS[j])] = jax.lax.dot_general(
            w_ref[...],
            xbuf[s, pl.ds(0, _CHUNKS[j]), :],
            dimension_numbers=(((1,), (1,)), ((), ())),
            preferred_element_type=jnp.float32,
        )
        out_copy(j).start()
        if j + _NSLOT < n:
            in_copy(j + _NSLOT).start()

    for j in range(max(n - _NSLOT, 0), n):
        out_copy(j).wait()


def kernel(x, gate_weight):
    M, K = x.shape
    E = gate_weight.shape[0]
    out_t = pl.pallas_call(
        _gate_body,
        in_specs=[
            pl.BlockSpec(memory_space=pl.ANY),
            pl.BlockSpec(memory_space=pltpu.VMEM),
        ],
        out_specs=pl.BlockSpec(memory_space=pl.ANY),
        out_shape=jax.ShapeDtypeStruct((E, M), jnp.float32),
        scratch_shapes=[
            pltpu.VMEM((_NSLOT, _MAXC, K), jnp.float32),
            pltpu.VMEM((_NSLOT, E, _MAXC), jnp.float32),
            pltpu.SemaphoreType.DMA((_NSLOT,)),
            pltpu.SemaphoreType.DMA((_NSLOT,)),
        ],
    )(x, gate_weight)
    return out_t.T


# speedup vs baseline: 1.0383x; 1.0383x over previous
import jax
import jax.numpy as jnp
from jax.experimental import pallas as pl

_BM = 1024


def _gate_body(x_ref, w_ref, o_ref):
    o_ref[...] = jax.lax.dot_general(
        w_ref[...],
        x_ref[...],
        dimension_numbers=(((1,), (1,)), ((), ())),
        preferred_element_type=jnp.float32,
    )


def kernel(x, gate_weight):
    M, K = x.shape
    E = gate_weight.shape[0]
    out_t = pl.pallas_call(
        _gate_body,
        grid=(M // _BM,),
        in_specs=[
            pl.BlockSpec((_BM, K), lambda i: (i, 0)),
            pl.BlockSpec((E, K), lambda i: (0, 0)),
        ],
        out_specs=pl.BlockSpec((E, _BM), lambda i: (0, i)),
        out_shape=jax.ShapeDtypeStruct((E, M), jnp.float32),
    )(x, gate_weight)
    return out_t.T
